# Initial kernel scaffold; baseline (speedup 1.0000x reference)
import functools

import jax
import jax.numpy as jnp
from jax import lax
from jax.experimental import pallas as pl
from jax.experimental.pallas import tpu as pltpu
from jax.experimental.pallas import tpu_sc as plsc

N = 10000
D = 128
DH = 64
E = 320000
NC = 2
NS = 16
NP = N + 16
RPS = NP // NS
CHUNK = 128
DEG_CHUNKS = 79
PROP_CHUNKS = 158
EP = NC * NS * DEG_CHUNKS * CHUNK

_mesh = plsc.VectorSubcoreMesh(core_axis_name="c", subcore_axis_name="s")


@functools.partial(
    pl.kernel,
    out_type=jax.ShapeDtypeStruct((NC, NP), jnp.float32),
    mesh=_mesh,
    scratch_types=[
        pltpu.VMEM((DEG_CHUNKS, CHUNK), jnp.int32),
        pltpu.VMEM((CHUNK,), jnp.float32),
        pltpu.VMEM_SHARED((NP,), jnp.float32),
    ],
)
def _deg_kernel(dst_hbm, zeros_hbm, ones_hbm, out_hbm, idx_v, ones_v, cnt_sh):
    c = lax.axis_index("c")
    s = lax.axis_index("s")
    wid = s * NC + c

    @pl.when(s == 0)
    def _():
        pltpu.sync_copy(zeros_hbm, cnt_sh)

    pltpu.sync_copy(ones_hbm, ones_v)
    pltpu.sync_copy(dst_hbm.at[wid], idx_v)
    plsc.subcore_barrier()

    @pl.loop(0, DEG_CHUNKS)
    def _(j):
        pltpu.sync_copy(ones_v, cnt_sh.at[idx_v.at[j]], add=True)

    plsc.subcore_barrier()

    @pl.when(s == 0)
    def _():
        pltpu.sync_copy(cnt_sh, out_hbm.at[c])


@functools.partial(
    pl.kernel,
    out_type=jax.ShapeDtypeStruct((NC, NP, DH), jnp.float32),
    mesh=_mesh,
    scratch_types=[
        pltpu.VMEM((PROP_CHUNKS, CHUNK), jnp.int32),
        pltpu.VMEM((PROP_CHUNKS, CHUNK), jnp.int32),
        pltpu.VMEM((CHUNK, DH), jnp.float32),
        pltpu.VMEM_SHARED((NP, DH), jnp.float32),
        pltpu.VMEM_SHARED((NP, DH), jnp.float32),
    ],
)
def _prop_kernel(y_hbm, src_hbm, dst_hbm, out_hbm, src_v, dst_v, buf, y_sh, acc_sh):
    c = lax.axis_index("c")
    s = lax.axis_index("s")
    r0 = s * RPS

    pltpu.sync_copy(y_hbm.at[c, pl.ds(r0, RPS)], y_sh.at[pl.ds(r0, RPS)])
    pltpu.sync_copy(y_hbm.at[c, pl.ds(r0, RPS)], acc_sh.at[pl.ds(r0, RPS)])
    pltpu.sync_copy(src_hbm.at[s], src_v)
    pltpu.sync_copy(dst_hbm.at[s], dst_v)
    plsc.subcore_barrier()

    @pl.loop(0, PROP_CHUNKS)
    def _(j):
        pltpu.sync_copy(y_sh.at[src_v.at[j]], buf)
        pltpu.sync_copy(buf, acc_sh.at[dst_v.at[j]], add=True)

    plsc.subcore_barrier()
    pltpu.sync_copy(acc_sh.at[pl.ds(r0, RPS)], out_hbm.at[c, pl.ds(r0, RPS)])


def _mm1_body(x_ref, w_ref, o_ref):
    xw = jnp.dot(x_ref[...], w_ref[...], preferred_element_type=jnp.float32,
                 precision=lax.Precision.HIGHEST)
    o_ref[0] = xw[:, :DH]
    o_ref[1] = xw[:, DH:]


def _scale_body(cnt_ref, xw_ref, dinv_ref, y_ref):
    deg = cnt_ref[0] + cnt_ref[1] + 1.0
    dinv = lax.rsqrt(deg)
    dinv_ref[...] = dinv
    y_ref[0] = xw_ref[0] * dinv
    y_ref[1] = xw_ref[1] * dinv


def _mid_body(acc_ref, dinv_ref, b1_ref, w2_ref, y2_ref):
    dinv = dinv_ref[...]
    h = jnp.concatenate([acc_ref[0], acc_ref[1]], axis=1) * dinv + b1_ref[...]
    h = jnp.maximum(h, 0.0)
    xw2 = jnp.dot(h, w2_ref[...], preferred_element_type=jnp.float32,
                  precision=lax.Precision.HIGHEST)
    y2_ref[0] = xw2[:, :DH] * dinv
    y2_ref[1] = xw2[:, DH:] * dinv


def _final_body(acc_ref, dinv_ref, b2_ref, o_ref):
    dinv = dinv_ref[...]
    z = jnp.concatenate([acc_ref[0], acc_ref[1]], axis=1) * dinv + b2_ref[...]
    z = z - jnp.max(z, axis=1, keepdims=True)
    o_ref[...] = z - jnp.log(jnp.sum(jnp.exp(z), axis=1, keepdims=True))


def kernel(x, edge_index, W1, b1, W2, b2):
    src = edge_index[0].astype(jnp.int32)
    dst = edge_index[1].astype(jnp.int32)
    pad = N + (jnp.arange(EP - E, dtype=jnp.int32) % 16)
    src_p = jnp.concatenate([src, pad]).reshape(NS, PROP_CHUNKS, CHUNK)
    dst_p = jnp.concatenate([dst, pad])
    dst_deg = dst_p.reshape(NC * NS, DEG_CHUNKS, CHUNK)
    dst_sc = dst_p.reshape(NS, PROP_CHUNKS, CHUNK)

    xpad = jnp.pad(x, ((0, NP - N), (0, 0)))
    zeros_np = jnp.zeros((NP,), jnp.float32)
    ones_c = jnp.ones((CHUNK,), jnp.float32)

    counts = _deg_kernel(dst_deg, zeros_np, ones_c)
    xw1 = pl.pallas_call(
        _mm1_body,
        out_shape=jax.ShapeDtypeStruct((NC, NP, DH), jnp.float32),
    )(xpad, W1)

    dinv, y1 = pl.pallas_call(
        _scale_body,
        out_shape=(jax.ShapeDtypeStruct((NP, 1), jnp.float32),
                   jax.ShapeDtypeStruct((NC, NP, DH), jnp.float32)),
    )(counts.reshape(NC, NP, 1), xw1)

    acc1 = _prop_kernel(y1, src_p, dst_sc)

    y2 = pl.pallas_call(
        _mid_body,
        out_shape=jax.ShapeDtypeStruct((NC, NP, DH), jnp.float32),
    )(acc1, dinv, b1.reshape(1, D), W2)

    acc2 = _prop_kernel(y2, src_p, dst_sc)

    z = pl.pallas_call(
        _final_body,
        out_shape=jax.ShapeDtypeStruct((NP, D), jnp.float32),
    )(acc2, dinv, b2.reshape(1, D))
    return z[:N]


# speedup vs baseline: 12.4228x; 12.4228x over previous
import functools

import jax
import jax.numpy as jnp
from jax import lax
from jax.experimental import pallas as pl
from jax.experimental.pallas import tpu as pltpu
from jax.experimental.pallas import tpu_sc as plsc

N = 10000
D = 128
E = 320000
NC = 2
NS = 16
NP = 10112
HN = NP // NC
CHUNK = 128
PROP_CHUNKS = 158
EP = NS * PROP_CHUNKS * CHUNK
WRS = 632

_mesh = plsc.VectorSubcoreMesh(core_axis_name="c", subcore_axis_name="s")


@functools.partial(
    pl.kernel,
    out_type=jax.ShapeDtypeStruct((NC, HN), jnp.float32),
    mesh=_mesh,
    scratch_types=[
        pltpu.VMEM((PROP_CHUNKS, CHUNK), jnp.int32),
        pltpu.VMEM((PROP_CHUNKS, CHUNK), jnp.float32),
        pltpu.VMEM_SHARED((HN,), jnp.float32),
    ],
)
def _deg_kernel(dst_hbm, val_hbm, zeros_hbm, out_hbm, idx_v, val_v, cnt_sh):
    c = lax.axis_index("c")
    s = lax.axis_index("s")

    @pl.when(s == 0)
    def _():
        pltpu.sync_copy(zeros_hbm, cnt_sh)

    pltpu.sync_copy(dst_hbm.at[c, s], idx_v)
    pltpu.sync_copy(val_hbm.at[c, s], val_v)
    plsc.subcore_barrier()

    @pl.loop(0, PROP_CHUNKS)
    def _(j):
        pltpu.sync_copy(val_v.at[j], cnt_sh.at[idx_v.at[j]], add=True)

    plsc.subcore_barrier()

    @pl.when(s == 0)
    def _():
        pltpu.sync_copy(cnt_sh, out_hbm.at[c])


@functools.partial(
    pl.kernel,
    out_type=jax.ShapeDtypeStruct((NP, D), jnp.float32),
    mesh=_mesh,
    scratch_types=[
        pltpu.VMEM((PROP_CHUNKS, CHUNK), jnp.int32),
        pltpu.VMEM((PROP_CHUNKS, CHUNK), jnp.int32),
        pltpu.VMEM((CHUNK, D), jnp.float32),
        pltpu.VMEM_SHARED((HN, D), jnp.float32),
    ],
)
def _prop_kernel(y_hbm, src_hbm, dst_hbm, out_hbm, src_v, dst_v, buf, acc_sh):
    c = lax.axis_index("c")
    s = lax.axis_index("s")

    @pl.when(s < 8)
    def _():
        r0 = s * WRS
        pltpu.sync_copy(y_hbm.at[pl.ds(c * HN + r0, WRS)], acc_sh.at[pl.ds(r0, WRS)])

    pltpu.sync_copy(src_hbm.at[c, s], src_v)
    pltpu.sync_copy(dst_hbm.at[c, s], dst_v)
    plsc.subcore_barrier()

    @pl.loop(0, PROP_CHUNKS)
    def _(j):
        pltpu.sync_copy(y_hbm.at[src_v.at[j]], buf)
        pltpu.sync_copy(buf, acc_sh.at[dst_v.at[j]], add=True)

    plsc.subcore_barrier()

    @pl.when(s < 8)
    def _():
        r0 = s * WRS
        pltpu.sync_copy(acc_sh.at[pl.ds(r0, WRS)], out_hbm.at[pl.ds(c * HN + r0, WRS)])


def _mm1_body(x_ref, w_ref, o_ref):
    o_ref[...] = jnp.dot(x_ref[...], w_ref[...], preferred_element_type=jnp.float32,
                         precision=lax.Precision.HIGHEST)


def _scale_body(cnt_ref, xw_ref, dinv_ref, y_ref):
    deg = cnt_ref[...] + 1.0
    dinv = lax.rsqrt(deg)
    dinv_ref[...] = dinv
    y_ref[...] = xw_ref[...] * dinv


def _mid_body(acc_ref, dinv_ref, b1_ref, w2_ref, y2_ref):
    dinv = dinv_ref[...]
    h = jnp.maximum(acc_ref[...] * dinv + b1_ref[...], 0.0)
    xw2 = jnp.dot(h, w2_ref[...], preferred_element_type=jnp.float32,
                  precision=lax.Precision.HIGHEST)
    y2_ref[...] = xw2 * dinv


def _final_body(acc_ref, dinv_ref, b2_ref, o_ref):
    z = acc_ref[...] * dinv_ref[...] + b2_ref[...]
    z = z - jnp.max(z, axis=1, keepdims=True)
    o_ref[...] = z - jnp.log(jnp.sum(jnp.exp(z), axis=1, keepdims=True))


def kernel(x, edge_index, W1, b1, W2, b2):
    src = edge_index[0].astype(jnp.int32)
    dst = edge_index[1].astype(jnp.int32)
    eid = jnp.arange(EP, dtype=jnp.int32)
    zrow = N + (eid % (NP - N))
    src_p = jnp.concatenate([src, zrow[E:]])
    dst_p = jnp.concatenate([dst, jnp.zeros((EP - E,), jnp.int32)])
    real = eid < E

    def route(c):
        own = real & (dst_p >= c * HN) & (dst_p < (c + 1) * HN)
        src_c = jnp.where(own, src_p, zrow)
        dst_c = jnp.where(own, dst_p - c * HN, eid % HN)
        val_c = jnp.where(own, 1.0, 0.0).astype(jnp.float32)
        return src_c, dst_c, val_c

    s0, d0, v0 = route(0)
    s1, d1, v1 = route(1)
    shape4 = (NC, NS, PROP_CHUNKS, CHUNK)
    src_sc = jnp.stack([s0, s1]).reshape(shape4)
    dst_sc = jnp.stack([d0, d1]).reshape(shape4)
    val_sc = jnp.stack([v0, v1]).reshape(shape4)

    xpad = jnp.pad(x, ((0, NP - N), (0, 0)))
    zeros_hn = jnp.zeros((HN,), jnp.float32)

    counts = _deg_kernel(dst_sc, val_sc, zeros_hn)
    xw1 = pl.pallas_call(
        _mm1_body,
        out_shape=jax.ShapeDtypeStruct((NP, D), jnp.float32),
    )(xpad, W1)

    dinv, y1 = pl.pallas_call(
        _scale_body,
        out_shape=(jax.ShapeDtypeStruct((NP, 1), jnp.float32),
                   jax.ShapeDtypeStruct((NP, D), jnp.float32)),
    )(counts.reshape(NP, 1), xw1)

    acc1 = _prop_kernel(y1, src_sc, dst_sc)

    y2 = pl.pallas_call(
        _mid_body,
        out_shape=jax.ShapeDtypeStruct((NP, D), jnp.float32),
    )(acc1, dinv, b1.reshape(1, D), W2)

    acc2 = _prop_kernel(y2, src_sc, dst_sc)

    z = pl.pallas_call(
        _final_body,
        out_shape=jax.ShapeDtypeStruct((NP, D), jnp.float32),
    )(acc2, dinv, b2.reshape(1, D))
    return z[:N]


# speedup vs baseline: 13.3176x; 1.0720x over previous
import functools

import jax
import jax.numpy as jnp
from jax import lax
from jax.experimental import pallas as pl
from jax.experimental.pallas import tpu as pltpu
from jax.experimental.pallas import tpu_sc as plsc

N = 10000
D = 128
E = 320000
NC = 2
NS = 16
NP = 10112
HN = NP // NC
CHUNK = 128
PROP_CHUNKS = 158
EP = NS * PROP_CHUNKS * CHUNK
WRS = 632

_mesh = plsc.VectorSubcoreMesh(core_axis_name="c", subcore_axis_name="s")


@functools.partial(
    pl.kernel,
    out_type=jax.ShapeDtypeStruct((NC, HN), jnp.float32),
    mesh=_mesh,
    scratch_types=[
        pltpu.VMEM((PROP_CHUNKS, CHUNK), jnp.int32),
        pltpu.VMEM((PROP_CHUNKS, CHUNK), jnp.float32),
        pltpu.VMEM_SHARED((HN,), jnp.float32),
    ],
)
def _deg_kernel(dst_hbm, val_hbm, zeros_hbm, out_hbm, idx_v, val_v, cnt_sh):
    c = lax.axis_index("c")
    s = lax.axis_index("s")

    @pl.when(s == 0)
    def _():
        pltpu.sync_copy(zeros_hbm, cnt_sh)

    pltpu.sync_copy(dst_hbm.at[c, s], idx_v)
    pltpu.sync_copy(val_hbm.at[c, s], val_v)
    plsc.subcore_barrier()

    @pl.loop(0, PROP_CHUNKS)
    def _(j):
        pltpu.sync_copy(val_v.at[j], cnt_sh.at[idx_v.at[j]], add=True)

    plsc.subcore_barrier()

    @pl.when(s == 0)
    def _():
        pltpu.sync_copy(cnt_sh, out_hbm.at[c])


@functools.partial(
    pl.kernel,
    out_type=jax.ShapeDtypeStruct((NP, D), jnp.float32),
    mesh=_mesh,
    scratch_types=[
        pltpu.VMEM((PROP_CHUNKS, CHUNK), jnp.int32),
        pltpu.VMEM((PROP_CHUNKS, CHUNK), jnp.int32),
        pltpu.VMEM((CHUNK, D), jnp.float32),
        pltpu.VMEM((CHUNK, D), jnp.float32),
        pltpu.VMEM_SHARED((HN, D), jnp.float32),
        pltpu.SemaphoreType.DMA,
        pltpu.SemaphoreType.DMA,
    ],
)
def _prop_kernel(y_hbm, src_hbm, dst_hbm, out_hbm,
                 src_v, dst_v, buf0, buf1, acc_sh, sem0, sem1):
    c = lax.axis_index("c")
    s = lax.axis_index("s")

    @pl.when(s < 8)
    def _():
        r0 = s * WRS
        pltpu.sync_copy(y_hbm.at[pl.ds(c * HN + r0, WRS)], acc_sh.at[pl.ds(r0, WRS)])

    pltpu.sync_copy(src_hbm.at[c, s], src_v)
    pltpu.sync_copy(dst_hbm.at[c, s], dst_v)
    plsc.subcore_barrier()

    def start(j, buf, sem):
        pltpu.async_copy(y_hbm.at[src_v.at[j]], buf, sem)

    def finish(j, buf, sem):
        pltpu.make_async_copy(y_hbm.at[src_v.at[j]], buf, sem).wait()
        pltpu.sync_copy(buf, acc_sh.at[dst_v.at[j]], add=True)

    start(0, buf0, sem0)

    @pl.loop(0, PROP_CHUNKS, step=2)
    def _(j):
        start(j + 1, buf1, sem1)
        finish(j, buf0, sem0)

        @pl.when(j + 2 < PROP_CHUNKS)
        def _():
            start(j + 2, buf0, sem0)

        finish(j + 1, buf1, sem1)

    plsc.subcore_barrier()

    @pl.when(s < 8)
    def _():
        r0 = s * WRS
        pltpu.sync_copy(acc_sh.at[pl.ds(r0, WRS)], out_hbm.at[pl.ds(c * HN + r0, WRS)])


def _mm1_body(x_ref, w_ref, o_ref):
    o_ref[...] = jnp.dot(x_ref[...], w_ref[...], preferred_element_type=jnp.float32,
                         precision=lax.Precision.HIGHEST)


def _scale_body(cnt_ref, xw_ref, dinv_ref, y_ref):
    deg = cnt_ref[...] + 1.0
    dinv = lax.rsqrt(deg)
    dinv_ref[...] = dinv
    y_ref[...] = xw_ref[...] * dinv


def _mid_body(acc_ref, dinv_ref, b1_ref, w2_ref, y2_ref):
    dinv = dinv_ref[...]
    h = jnp.maximum(acc_ref[...] * dinv + b1_ref[...], 0.0)
    xw2 = jnp.dot(h, w2_ref[...], preferred_element_type=jnp.float32,
                  precision=lax.Precision.HIGHEST)
    y2_ref[...] = xw2 * dinv


def _final_body(acc_ref, dinv_ref, b2_ref, o_ref):
    z = acc_ref[...] * dinv_ref[...] + b2_ref[...]
    z = z - jnp.max(z, axis=1, keepdims=True)
    o_ref[...] = z - jnp.log(jnp.sum(jnp.exp(z), axis=1, keepdims=True))


def kernel(x, edge_index, W1, b1, W2, b2):
    src = edge_index[0].astype(jnp.int32)
    dst = edge_index[1].astype(jnp.int32)
    eid = jnp.arange(EP, dtype=jnp.int32)
    zrow = N + (eid % (NP - N))
    src_p = jnp.concatenate([src, zrow[E:]])
    dst_p = jnp.concatenate([dst, jnp.zeros((EP - E,), jnp.int32)])
    real = eid < E

    def route(c):
        own = real & (dst_p >= c * HN) & (dst_p < (c + 1) * HN)
        src_c = jnp.where(own, src_p, zrow)
        dst_c = jnp.where(own, dst_p - c * HN, eid % HN)
        val_c = jnp.where(own, 1.0, 0.0).astype(jnp.float32)
        return src_c, dst_c, val_c

    s0, d0, v0 = route(0)
    s1, d1, v1 = route(1)
    shape4 = (NC, NS, PROP_CHUNKS, CHUNK)
    src_sc = jnp.stack([s0, s1]).reshape(shape4)
    dst_sc = jnp.stack([d0, d1]).reshape(shape4)
    val_sc = jnp.stack([v0, v1]).reshape(shape4)

    xpad = jnp.pad(x, ((0, NP - N), (0, 0)))
    zeros_hn = jnp.zeros((HN,), jnp.float32)

    counts = _deg_kernel(dst_sc, val_sc, zeros_hn)
    xw1 = pl.pallas_call(
        _mm1_body,
        out_shape=jax.ShapeDtypeStruct((NP, D), jnp.float32),
    )(xpad, W1)

    dinv, y1 = pl.pallas_call(
        _scale_body,
        out_shape=(jax.ShapeDtypeStruct((NP, 1), jnp.float32),
                   jax.ShapeDtypeStruct((NP, D), jnp.float32)),
    )(counts.reshape(NP, 1), xw1)

    acc1 = _prop_kernel(y1, src_sc, dst_sc)

    y2 = pl.pallas_call(
        _mid_body,
        out_shape=jax.ShapeDtypeStruct((NP, D), jnp.float32),
    )(acc1, dinv, b1.reshape(1, D), W2)

    acc2 = _prop_kernel(y2, src_sc, dst_sc)

    z = pl.pallas_call(
        _final_body,
        out_shape=jax.ShapeDtypeStruct((NP, D), jnp.float32),
    )(acc2, dinv, b2.reshape(1, D))
    return z[:N]


# speedup vs baseline: 20.4706x; 1.5371x over previous
import functools

import jax
import jax.numpy as jnp
from jax import lax
from jax.experimental import pallas as pl
from jax.experimental.pallas import tpu as pltpu
from jax.experimental.pallas import tpu_sc as plsc

N = 10000
D = 128
E = 320000
NC = 2
NS = 16
NP = 10112
HN = NP // NC
CHUNK = 128
PROP_CHUNKS = 158
EP = NS * PROP_CHUNKS * CHUNK
WRS = 632
DUM = 1024

_mesh = plsc.VectorSubcoreMesh(core_axis_name="c", subcore_axis_name="s")


@functools.partial(
    pl.kernel,
    out_type=jax.ShapeDtypeStruct((NC, HN), jnp.float32),
    mesh=_mesh,
    scratch_types=[
        pltpu.VMEM((PROP_CHUNKS, CHUNK), jnp.int32),
        pltpu.VMEM((PROP_CHUNKS, CHUNK), jnp.float32),
        pltpu.VMEM_SHARED((HN,), jnp.float32),
    ],
)
def _deg_kernel(dst_hbm, val_hbm, zeros_hbm, out_hbm, idx_v, val_v, cnt_sh):
    c = lax.axis_index("c")
    s = lax.axis_index("s")

    @pl.when(s == 0)
    def _():
        pltpu.sync_copy(zeros_hbm, cnt_sh)

    pltpu.sync_copy(dst_hbm.at[c, s], idx_v)
    pltpu.sync_copy(val_hbm.at[c, s], val_v)
    plsc.subcore_barrier()

    @pl.loop(0, PROP_CHUNKS)
    def _(j):
        pltpu.sync_copy(val_v.at[j], cnt_sh.at[idx_v.at[j]], add=True)

    plsc.subcore_barrier()

    @pl.when(s == 0)
    def _():
        pltpu.sync_copy(cnt_sh, out_hbm.at[c])


@functools.partial(
    pl.kernel,
    out_type=jax.ShapeDtypeStruct((NP, D), jnp.float32),
    mesh=_mesh,
    scratch_types=[
        pltpu.VMEM((PROP_CHUNKS, CHUNK), jnp.int32),
        pltpu.VMEM((PROP_CHUNKS, CHUNK), jnp.int32),
        pltpu.VMEM((CHUNK, D), jnp.float32),
        pltpu.VMEM((CHUNK, D), jnp.float32),
        pltpu.VMEM_SHARED((HN + DUM, D), jnp.float32),
        pltpu.SemaphoreType.DMA,
        pltpu.SemaphoreType.DMA,
    ],
)
def _prop_kernel(y_hbm, src_hbm, dst_hbm, out_hbm,
                 src_v, dst_v, buf0, buf1, acc_sh, sem0, sem1):
    c = lax.axis_index("c")
    s = lax.axis_index("s")

    @pl.when(s < 8)
    def _():
        r0 = s * WRS
        pltpu.sync_copy(y_hbm.at[pl.ds(c * HN + r0, WRS)], acc_sh.at[pl.ds(r0, WRS)])

    pltpu.sync_copy(src_hbm.at[c, s], src_v)
    pltpu.sync_copy(dst_hbm.at[c, s], dst_v)
    plsc.subcore_barrier()

    def start(j, buf, sem):
        pltpu.async_copy(y_hbm.at[src_v.at[j]], buf, sem)

    def finish(j, buf, sem):
        pltpu.make_async_copy(y_hbm.at[src_v.at[j]], buf, sem).wait()
        pltpu.sync_copy(buf, acc_sh.at[dst_v.at[j]], add=True)

    start(0, buf0, sem0)

    @pl.loop(0, PROP_CHUNKS, step=2)
    def _(j):
        start(j + 1, buf1, sem1)
        finish(j, buf0, sem0)

        @pl.when(j + 2 < PROP_CHUNKS)
        def _():
            start(j + 2, buf0, sem0)

        finish(j + 1, buf1, sem1)

    plsc.subcore_barrier()

    @pl.when(s < 8)
    def _():
        r0 = s * WRS
        pltpu.sync_copy(acc_sh.at[pl.ds(r0, WRS)], out_hbm.at[pl.ds(c * HN + r0, WRS)])


def _mm1_body(x_ref, w_ref, o_ref):
    o_ref[...] = jnp.dot(x_ref[...], w_ref[...], preferred_element_type=jnp.float32,
                         precision=lax.Precision.HIGHEST)


def _scale_body(cnt_ref, xw_ref, dinv_ref, y_ref):
    deg = cnt_ref[...] + 1.0
    dinv = lax.rsqrt(deg)
    dinv_ref[...] = dinv
    y_ref[...] = xw_ref[...] * dinv


def _mid_body(acc_ref, dinv_ref, b1_ref, w2_ref, y2_ref):
    dinv = dinv_ref[...]
    h = jnp.maximum(acc_ref[...] * dinv + b1_ref[...], 0.0)
    xw2 = jnp.dot(h, w2_ref[...], preferred_element_type=jnp.float32,
                  precision=lax.Precision.HIGHEST)
    y2_ref[...] = xw2 * dinv


def _final_body(acc_ref, dinv_ref, b2_ref, o_ref):
    z = acc_ref[...] * dinv_ref[...] + b2_ref[...]
    z = z - jnp.max(z, axis=1, keepdims=True)
    o_ref[...] = z - jnp.log(jnp.sum(jnp.exp(z), axis=1, keepdims=True))


def kernel(x, edge_index, W1, b1, W2, b2):
    src = edge_index[0].astype(jnp.int32)
    dst = edge_index[1].astype(jnp.int32)
    eid = jnp.arange(EP, dtype=jnp.int32)
    src_p = jnp.concatenate([src, eid[E:] % N])
    dst_p = jnp.concatenate([dst, jnp.zeros((EP - E,), jnp.int32)])
    real = eid < E

    def route(c):
        own = real & (dst_p >= c * HN) & (dst_p < (c + 1) * HN)
        dst_c = jnp.where(own, dst_p - c * HN, HN + eid % DUM)
        dstdeg_c = jnp.where(own, dst_p - c * HN, eid % HN)
        val_c = jnp.where(own, 1.0, 0.0).astype(jnp.float32)
        return dst_c, dstdeg_c, val_c

    d0, g0, v0 = route(0)
    d1, g1, v1 = route(1)
    shape4 = (NC, NS, PROP_CHUNKS, CHUNK)
    src_sc = jnp.broadcast_to(src_p.reshape((1,) + shape4[1:]), shape4)
    dst_sc = jnp.stack([d0, d1]).reshape(shape4)
    deg_sc = jnp.stack([g0, g1]).reshape(shape4)
    val_sc = jnp.stack([v0, v1]).reshape(shape4)

    xpad = jnp.pad(x, ((0, NP - N), (0, 0)))
    zeros_hn = jnp.zeros((HN,), jnp.float32)

    counts = _deg_kernel(deg_sc, val_sc, zeros_hn)
    xw1 = pl.pallas_call(
        _mm1_body,
        out_shape=jax.ShapeDtypeStruct((NP, D), jnp.float32),
    )(xpad, W1)

    dinv, y1 = pl.pallas_call(
        _scale_body,
        out_shape=(jax.ShapeDtypeStruct((NP, 1), jnp.float32),
                   jax.ShapeDtypeStruct((NP, D), jnp.float32)),
    )(counts.reshape(NP, 1), xw1)

    acc1 = _prop_kernel(y1, src_sc, dst_sc)

    y2 = pl.pallas_call(
        _mid_body,
        out_shape=jax.ShapeDtypeStruct((NP, D), jnp.float32),
    )(acc1, dinv, b1.reshape(1, D), W2)

    acc2 = _prop_kernel(y2, src_sc, dst_sc)

    z = pl.pallas_call(
        _final_body,
        out_shape=jax.ShapeDtypeStruct((NP, D), jnp.float32),
    )(acc2, dinv, b2.reshape(1, D))
    return z[:N]


# speedup vs baseline: 31.3844x; 1.5331x over previous
import dataclasses
import functools

import jax
import jax.numpy as jnp
from jax import lax
from jax.experimental import pallas as pl
from jax.experimental.pallas import tpu as pltpu
from jax.experimental.pallas import tpu_sc as plsc

N = 10000
D = 128
E = 320000
NC = 2
NS = 16
NP = 10112
HN = NP // NC
CHUNK = 128
NCH = 158
ECS = NCH * CHUNK
EP = NS * ECS
WRS = 632
DUM = 1024
L = 16

_mesh = plsc.VectorSubcoreMesh(core_axis_name="c", subcore_axis_name="s")

_prep_cp = pltpu.CompilerParams()
if "needs_layout_passes" in pltpu.CompilerParams.__dataclass_fields__:
    _prep_cp = dataclasses.replace(_prep_cp, needs_layout_passes=False)


@functools.partial(
    pl.kernel,
    out_type=(jax.ShapeDtypeStruct((NC, NS, ECS), jnp.int32),
              jax.ShapeDtypeStruct((NC, NS, ECS), jnp.int32),
              jax.ShapeDtypeStruct((NC, NS, L), jnp.int32),
              jax.ShapeDtypeStruct((NC, NS, HN), jnp.float32)),
    mesh=_mesh,
    compiler_params=_prep_cp,
    scratch_types=[
        pltpu.VMEM((ECS,), jnp.int32),
        pltpu.VMEM((ECS,), jnp.int32),
        pltpu.VMEM((ECS + L,), jnp.int32),
        pltpu.VMEM((ECS + L,), jnp.int32),
        pltpu.VMEM((HN,), jnp.float32),
        pltpu.VMEM((L,), jnp.int32),
    ],
)
def _prep_kernel(src_hbm, dst_hbm,
                 osrc_hbm, odst_hbm, ocnt_hbm, odeg_hbm,
                 src_v, dst_v, csrc_v, cdst_v, hist_v, cnt_v):
    c = lax.axis_index("c")
    s = lax.axis_index("s")
    lo = c * HN

    pltpu.sync_copy(src_hbm.at[s], src_v)
    pltpu.sync_copy(dst_hbm.at[s], dst_v)

    base = jax.lax.iota(jnp.int32, L)
    onesv = jnp.ones((L,), jnp.float32)

    @pl.loop(0, ECS + L, step=L)
    def _(i):
        v = base + i
        csrc_v[pl.ds(i, L)] = v & 8191
        cdst_v[pl.ds(i, L)] = HN + (v & (DUM - 1))

    @pl.loop(0, HN, step=L)
    def _(i):
        hist_v[pl.ds(i, L)] = jnp.zeros((L,), jnp.float32)

    def body(i, o):
        d = dst_v[pl.ds(i * L, L)]
        sr = src_v[pl.ds(i * L, L)]
        own = (d >= lo) & (d < lo + HN)
        dl = jnp.where(own, d - lo, 0)
        pref = plsc.cumsum(jnp.where(own, 1, 0))
        pos = jnp.where(own, o + pref - 1, 0)
        plsc.store_scatter(csrc_v, [pos], sr, mask=own)
        plsc.store_scatter(cdst_v, [pos], dl, mask=own)
        plsc.addupdate_scatter(hist_v, [dl], onesv, mask=own)
        return o + jnp.max(pref)

    count = lax.fori_loop(0, ECS // L, body, jnp.int32(0))
    tc = (count + (CHUNK - 1)) >> 7
    cnt_v[...] = jnp.broadcast_to(tc, (L,))

    pltpu.sync_copy(cnt_v, ocnt_hbm.at[c, s])
    pltpu.sync_copy(csrc_v.at[pl.ds(0, ECS)], osrc_hbm.at[c, s])
    pltpu.sync_copy(cdst_v.at[pl.ds(0, ECS)], odst_hbm.at[c, s])
    pltpu.sync_copy(hist_v, odeg_hbm.at[c, s])


@functools.partial(
    pl.kernel,
    out_type=jax.ShapeDtypeStruct((NP, D), jnp.float32),
    mesh=_mesh,
    scratch_types=[
        pltpu.VMEM((NCH, CHUNK), jnp.int32),
        pltpu.VMEM((NCH, CHUNK), jnp.int32),
        pltpu.VMEM((CHUNK, D), jnp.float32),
        pltpu.VMEM((CHUNK, D), jnp.float32),
        pltpu.VMEM_SHARED((HN + DUM, D), jnp.float32),
        pltpu.VMEM((L,), jnp.int32),
        pltpu.SemaphoreType.DMA,
        pltpu.SemaphoreType.DMA,
    ],
)
def _prop_kernel(y_hbm, src_hbm, dst_hbm, cnt_hbm, out_hbm,
                 src_v, dst_v, buf0, buf1, acc_sh, cnt_v, sem0, sem1):
    c = lax.axis_index("c")
    s = lax.axis_index("s")

    @pl.when(s < 8)
    def _():
        r0 = s * WRS
        pltpu.sync_copy(y_hbm.at[pl.ds(c * HN + r0, WRS)], acc_sh.at[pl.ds(r0, WRS)])

    pltpu.sync_copy(src_hbm.at[c, s], src_v)
    pltpu.sync_copy(dst_hbm.at[c, s], dst_v)
    pltpu.sync_copy(cnt_hbm.at[c, s], cnt_v)
    plsc.subcore_barrier()
    tc = cnt_v[...][0]

    def start(j, buf, sem):
        pltpu.async_copy(y_hbm.at[src_v.at[j]], buf, sem)

    def finish(j, buf, sem):
        pltpu.make_async_copy(y_hbm.at[src_v.at[j]], buf, sem).wait()
        pltpu.sync_copy(buf, acc_sh.at[dst_v.at[j]], add=True)

    @pl.when(tc > 0)
    def _():
        start(0, buf0, sem0)

    @pl.loop(0, NCH, step=2)
    def _(j):
        @pl.when(j < tc)
        def _():
            @pl.when(j + 1 < tc)
            def _():
                start(j + 1, buf1, sem1)

            finish(j, buf0, sem0)

            @pl.when(j + 2 < tc)
            def _():
                start(j + 2, buf0, sem0)

            @pl.when(j + 1 < tc)
            def _():
                finish(j + 1, buf1, sem1)

    plsc.subcore_barrier()

    @pl.when(s < 8)
    def _():
        r0 = s * WRS
        pltpu.sync_copy(acc_sh.at[pl.ds(r0, WRS)], out_hbm.at[pl.ds(c * HN + r0, WRS)])


def _mm1_body(x_ref, w_ref, o_ref):
    o_ref[...] = jnp.dot(x_ref[...], w_ref[...], preferred_element_type=jnp.float32,
                         precision=lax.Precision.HIGHEST)


def _scale_body(cnt_ref, xw_ref, dinv_ref, y_ref):
    deg = jnp.sum(cnt_ref[...], axis=1, keepdims=True) + 1.0
    dinv = lax.rsqrt(deg)
    dinv_ref[...] = dinv
    y_ref[...] = xw_ref[...] * dinv


def _mid_body(acc_ref, dinv_ref, b1_ref, w2_ref, y2_ref):
    dinv = dinv_ref[...]
    h = jnp.maximum(acc_ref[...] * dinv + b1_ref[...], 0.0)
    xw2 = jnp.dot(h, w2_ref[...], preferred_element_type=jnp.float32,
                  precision=lax.Precision.HIGHEST)
    y2_ref[...] = xw2 * dinv


def _final_body(acc_ref, dinv_ref, b2_ref, o_ref):
    z = acc_ref[...] * dinv_ref[...] + b2_ref[...]
    z = z - jnp.max(z, axis=1, keepdims=True)
    o_ref[...] = z - jnp.log(jnp.sum(jnp.exp(z), axis=1, keepdims=True))


def kernel(x, edge_index, W1, b1, W2, b2):
    src = edge_index[0].astype(jnp.int32)
    dst = edge_index[1].astype(jnp.int32)
    src_flat = jnp.concatenate([src, jnp.zeros((EP - E,), jnp.int32)])
    dst_flat = jnp.concatenate([dst, jnp.full((EP - E,), NP, jnp.int32)])
    src_flat = src_flat.reshape(NS, ECS)
    dst_flat = dst_flat.reshape(NS, ECS)

    xpad = jnp.pad(x, ((0, NP - N), (0, 0)))

    osrc, odst, ocnt, odeg = _prep_kernel(src_flat, dst_flat)
    osrc = osrc.reshape(NC, NS, NCH, CHUNK)
    odst = odst.reshape(NC, NS, NCH, CHUNK)
    counts = jnp.transpose(odeg, (0, 2, 1)).reshape(NP, NS)

    xw1 = pl.pallas_call(
        _mm1_body,
        out_shape=jax.ShapeDtypeStruct((NP, D), jnp.float32),
    )(xpad, W1)

    dinv, y1 = pl.pallas_call(
        _scale_body,
        out_shape=(jax.ShapeDtypeStruct((NP, 1), jnp.float32),
                   jax.ShapeDtypeStruct((NP, D), jnp.float32)),
    )(counts, xw1)

    acc1 = _prop_kernel(y1, osrc, odst, ocnt)

    y2 = pl.pallas_call(
        _mid_body,
        out_shape=jax.ShapeDtypeStruct((NP, D), jnp.float32),
    )(acc1, dinv, b1.reshape(1, D), W2)

    acc2 = _prop_kernel(y2, osrc, odst, ocnt)

    z = pl.pallas_call(
        _final_body,
        out_shape=jax.ShapeDtypeStruct((NP, D), jnp.float32),
    )(acc2, dinv, b2.reshape(1, D))
    return z[:N]


# speedup vs baseline: 31.7980x; 1.0132x over previous
import dataclasses
import functools

import jax
import jax.numpy as jnp
from jax import lax
from jax.experimental import pallas as pl
from jax.experimental.pallas import tpu as pltpu
from jax.experimental.pallas import tpu_sc as plsc

N = 10000
D = 128
E = 320000
NC = 2
NS = 16
NP = 10112
HN = NP // NC
CHUNK = 128
NCH = 158
ECS = NCH * CHUNK
EP = NS * ECS
WRS = 632
DUM = 1024
L = 16

_mesh = plsc.VectorSubcoreMesh(core_axis_name="c", subcore_axis_name="s")

_prep_cp = pltpu.CompilerParams()
if "needs_layout_passes" in pltpu.CompilerParams.__dataclass_fields__:
    _prep_cp = dataclasses.replace(_prep_cp, needs_layout_passes=False)


@functools.partial(
    pl.kernel,
    out_type=(jax.ShapeDtypeStruct((NC, NS, ECS), jnp.int32),
              jax.ShapeDtypeStruct((NC, NS, ECS), jnp.int32),
              jax.ShapeDtypeStruct((NC, NS, L), jnp.int32),
              jax.ShapeDtypeStruct((NC, NS, HN), jnp.float32)),
    mesh=_mesh,
    compiler_params=_prep_cp,
    scratch_types=[
        pltpu.VMEM((ECS,), jnp.int32),
        pltpu.VMEM((ECS,), jnp.int32),
        pltpu.VMEM((ECS + L,), jnp.int32),
        pltpu.VMEM((ECS + L,), jnp.int32),
        pltpu.VMEM((HN,), jnp.float32),
        pltpu.VMEM((L,), jnp.int32),
    ],
)
def _prep_kernel(src_hbm, dst_hbm,
                 osrc_hbm, odst_hbm, ocnt_hbm, odeg_hbm,
                 src_v, dst_v, csrc_v, cdst_v, hist_v, cnt_v):
    c = lax.axis_index("c")
    s = lax.axis_index("s")
    lo = c * HN

    pltpu.sync_copy(src_hbm.at[s], src_v)
    pltpu.sync_copy(dst_hbm.at[s], dst_v)

    base = jax.lax.iota(jnp.int32, L)
    onesv = jnp.ones((L,), jnp.float32)

    @pl.loop(0, ECS + L, step=L)
    def _(i):
        v = base + i
        csrc_v[pl.ds(i, L)] = v & 8191
        cdst_v[pl.ds(i, L)] = HN + (v & (DUM - 1))

    @pl.loop(0, HN, step=L)
    def _(i):
        hist_v[pl.ds(i, L)] = jnp.zeros((L,), jnp.float32)

    def body(i, o):
        d = dst_v[pl.ds(i * L, L)]
        sr = src_v[pl.ds(i * L, L)]
        own = (d >= lo) & (d < lo + HN)
        dl = jnp.where(own, d - lo, 0)
        pref = plsc.cumsum(jnp.where(own, 1, 0))
        pos = jnp.where(own, o + pref - 1, 0)
        plsc.store_scatter(csrc_v, [pos], sr, mask=own)
        plsc.store_scatter(cdst_v, [pos], dl, mask=own)
        plsc.addupdate_scatter(hist_v, [dl], onesv, mask=own)
        return o + jnp.max(pref)

    count = lax.fori_loop(0, ECS // L, body, jnp.int32(0))
    tc = (count + (CHUNK - 1)) >> 7
    cnt_v[...] = jnp.broadcast_to(tc, (L,))

    pltpu.sync_copy(cnt_v, ocnt_hbm.at[c, s])
    pltpu.sync_copy(csrc_v.at[pl.ds(0, ECS)], osrc_hbm.at[c, s])
    pltpu.sync_copy(cdst_v.at[pl.ds(0, ECS)], odst_hbm.at[c, s])
    pltpu.sync_copy(hist_v, odeg_hbm.at[c, s])


@functools.partial(
    pl.kernel,
    out_type=jax.ShapeDtypeStruct((NP, D), jnp.float32),
    mesh=_mesh,
    scratch_types=[
        pltpu.VMEM((NCH, CHUNK), jnp.int32),
        pltpu.VMEM((NCH, CHUNK), jnp.int32),
        pltpu.VMEM((CHUNK, D), jnp.float32),
        pltpu.VMEM((CHUNK, D), jnp.float32),
        pltpu.VMEM_SHARED((HN + DUM, D), jnp.float32),
        pltpu.VMEM((L,), jnp.int32),
        pltpu.SemaphoreType.DMA,
        pltpu.SemaphoreType.DMA,
    ],
)
def _prop_kernel(y_hbm, src_hbm, dst_hbm, cnt_hbm, out_hbm,
                 src_v, dst_v, buf0, buf1, acc_sh, cnt_v, sem0, sem1):
    c = lax.axis_index("c")
    s = lax.axis_index("s")

    @pl.when(s < 8)
    def _():
        r0 = s * WRS
        pltpu.sync_copy(y_hbm.at[pl.ds(c * HN + r0, WRS)], acc_sh.at[pl.ds(r0, WRS)])

    pltpu.sync_copy(src_hbm.at[c, s], src_v)
    pltpu.sync_copy(dst_hbm.at[c, s], dst_v)
    pltpu.sync_copy(cnt_hbm.at[c, s], cnt_v)
    plsc.subcore_barrier()
    tc = cnt_v[...][0]

    def start(j, buf, sem):
        pltpu.async_copy(y_hbm.at[src_v.at[j]], buf, sem)

    def finish(j, buf, sem):
        pltpu.make_async_copy(y_hbm.at[src_v.at[j]], buf, sem).wait()
        pltpu.sync_copy(buf, acc_sh.at[dst_v.at[j]], add=True)

    @pl.when(tc > 0)
    def _():
        start(0, buf0, sem0)

    @pl.loop(0, NCH, step=2)
    def _(j):
        @pl.when(j < tc)
        def _():
            @pl.when(j + 1 < tc)
            def _():
                start(j + 1, buf1, sem1)

            finish(j, buf0, sem0)

            @pl.when(j + 2 < tc)
            def _():
                start(j + 2, buf0, sem0)

            @pl.when(j + 1 < tc)
            def _():
                finish(j + 1, buf1, sem1)

    plsc.subcore_barrier()

    @pl.when(s < 8)
    def _():
        r0 = s * WRS
        pltpu.sync_copy(acc_sh.at[pl.ds(r0, WRS)], out_hbm.at[pl.ds(c * HN + r0, WRS)])


def _mm1_body(x_ref, w_ref, o_ref):
    o_ref[...] = jnp.dot(x_ref[...], w_ref[...], preferred_element_type=jnp.float32,
                         precision=lax.Precision.HIGHEST)


def _scale_body(cnt_ref, xw_ref, dinv_ref, y_ref):
    deg = jnp.sum(cnt_ref[...], axis=1, keepdims=True) + 1.0
    dinv = lax.rsqrt(deg)
    dinv_ref[...] = dinv
    y_ref[...] = xw_ref[...] * dinv


def _mid_body(acc_ref, dinv_ref, b1_ref, w2_ref, y2_ref):
    dinv = dinv_ref[...]
    h = jnp.maximum(acc_ref[...] * dinv + b1_ref[...], 0.0)
    xw2 = jnp.dot(h, w2_ref[...], preferred_element_type=jnp.float32,
                  precision=lax.Precision.HIGHEST)
    y2_ref[...] = xw2 * dinv


def _final_body(acc_ref, dinv_ref, b2_ref, o_ref):
    z = acc_ref[...][:N] * dinv_ref[...][:N] + b2_ref[...]
    z = z - jnp.max(z, axis=1, keepdims=True)
    o_ref[...] = z - jnp.log(jnp.sum(jnp.exp(z), axis=1, keepdims=True))


def kernel(x, edge_index, W1, b1, W2, b2):
    src = edge_index[0].astype(jnp.int32)
    dst = edge_index[1].astype(jnp.int32)
    src_flat = jnp.concatenate([src, jnp.zeros((EP - E,), jnp.int32)])
    dst_flat = jnp.concatenate([dst, jnp.full((EP - E,), NP, jnp.int32)])
    src_flat = src_flat.reshape(NS, ECS)
    dst_flat = dst_flat.reshape(NS, ECS)

    xpad = jnp.pad(x, ((0, NP - N), (0, 0)))

    osrc, odst, ocnt, odeg = _prep_kernel(src_flat, dst_flat)
    osrc = osrc.reshape(NC, NS, NCH, CHUNK)
    odst = odst.reshape(NC, NS, NCH, CHUNK)
    counts = jnp.transpose(odeg, (0, 2, 1)).reshape(NP, NS)

    xw1 = pl.pallas_call(
        _mm1_body,
        out_shape=jax.ShapeDtypeStruct((NP, D), jnp.float32),
    )(xpad, W1)

    dinv, y1 = pl.pallas_call(
        _scale_body,
        out_shape=(jax.ShapeDtypeStruct((NP, 1), jnp.float32),
                   jax.ShapeDtypeStruct((NP, D), jnp.float32)),
    )(counts, xw1)

    acc1 = _prop_kernel(y1, osrc, odst, ocnt)

    y2 = pl.pallas_call(
        _mid_body,
        out_shape=jax.ShapeDtypeStruct((NP, D), jnp.float32),
    )(acc1, dinv, b1.reshape(1, D), W2)

    acc2 = _prop_kernel(y2, osrc, odst, ocnt)

    return pl.pallas_call(
        _final_body,
        out_shape=jax.ShapeDtypeStruct((N, D), jnp.float32),
    )(acc2, dinv, b2.reshape(1, D))


# speedup vs baseline: 32.0463x; 1.0078x over previous
import dataclasses
import functools

import jax
import jax.numpy as jnp
from jax import lax
from jax.experimental import pallas as pl
from jax.experimental.pallas import tpu as pltpu
from jax.experimental.pallas import tpu_sc as plsc

N = 10000
D = 128
E = 320000
NC = 2
NS = 16
NP = 10112
HN = NP // NC
CHUNK = 128
NCH = 158
ECS = NCH * CHUNK
EP = NS * ECS
WRS = 632
DUM = 1024
L = 16

_mesh = plsc.VectorSubcoreMesh(core_axis_name="c", subcore_axis_name="s")

_prep_cp = pltpu.CompilerParams()
if "needs_layout_passes" in pltpu.CompilerParams.__dataclass_fields__:
    _prep_cp = dataclasses.replace(_prep_cp, needs_layout_passes=False)


@functools.partial(
    pl.kernel,
    out_type=(jax.ShapeDtypeStruct((NC, NS, ECS), jnp.int32),
              jax.ShapeDtypeStruct((NC, NS, ECS), jnp.int32),
              jax.ShapeDtypeStruct((NC, NS, L), jnp.int32),
              jax.ShapeDtypeStruct((NC, NS, HN), jnp.float32)),
    mesh=_mesh,
    compiler_params=_prep_cp,
    scratch_types=[
        pltpu.VMEM((ECS,), jnp.int32),
        pltpu.VMEM((ECS,), jnp.int32),
        pltpu.VMEM((ECS + L,), jnp.int32),
        pltpu.VMEM((ECS + L,), jnp.int32),
        pltpu.VMEM((HN,), jnp.float32),
        pltpu.VMEM((L,), jnp.int32),
    ],
)
def _prep_kernel(src_hbm, dst_hbm,
                 osrc_hbm, odst_hbm, ocnt_hbm, odeg_hbm,
                 src_v, dst_v, csrc_v, cdst_v, hist_v, cnt_v):
    c = lax.axis_index("c")
    s = lax.axis_index("s")
    lo = c * HN

    pltpu.sync_copy(src_hbm.at[s], src_v)
    pltpu.sync_copy(dst_hbm.at[s], dst_v)

    base = jax.lax.iota(jnp.int32, L)
    onesv = jnp.ones((L,), jnp.float32)

    @pl.loop(0, ECS + L, step=L)
    def _(i):
        v = base + i
        csrc_v[pl.ds(i, L)] = v & 8191
        cdst_v[pl.ds(i, L)] = HN + (v & (DUM - 1))

    @pl.loop(0, HN, step=L)
    def _(i):
        hist_v[pl.ds(i, L)] = jnp.zeros((L,), jnp.float32)

    def body(i, o):
        d = dst_v[pl.ds(i * L, L)]
        sr = src_v[pl.ds(i * L, L)]
        own = (d >= lo) & (d < lo + HN)
        dl = jnp.where(own, d - lo, 0)
        pref = plsc.cumsum(jnp.where(own, 1, 0))
        pos = jnp.where(own, o + pref - 1, 0)
        plsc.store_scatter(csrc_v, [pos], sr, mask=own)
        plsc.store_scatter(cdst_v, [pos], dl, mask=own)
        plsc.addupdate_scatter(hist_v, [dl], onesv, mask=own)
        return o + jnp.max(pref)

    count = lax.fori_loop(0, ECS // L, body, jnp.int32(0))
    tc = (count + (CHUNK - 1)) >> 7
    cnt_v[...] = jnp.broadcast_to(tc, (L,))

    pltpu.sync_copy(cnt_v, ocnt_hbm.at[c, s])
    pltpu.sync_copy(csrc_v.at[pl.ds(0, ECS)], osrc_hbm.at[c, s])
    pltpu.sync_copy(cdst_v.at[pl.ds(0, ECS)], odst_hbm.at[c, s])
    pltpu.sync_copy(hist_v, odeg_hbm.at[c, s])


@functools.partial(
    pl.kernel,
    out_type=jax.ShapeDtypeStruct((NP, D), jnp.float32),
    mesh=_mesh,
    scratch_types=[
        pltpu.VMEM((NCH, CHUNK), jnp.int32),
        pltpu.VMEM((NCH, CHUNK), jnp.int32),
        pltpu.VMEM((CHUNK, D), jnp.float32),
        pltpu.VMEM((CHUNK, D), jnp.float32),
        pltpu.VMEM_SHARED((HN + DUM, D), jnp.float32),
        pltpu.VMEM((L,), jnp.int32),
        pltpu.SemaphoreType.DMA,
        pltpu.SemaphoreType.DMA,
    ],
)
def _prop_kernel(y_hbm, src_hbm, dst_hbm, cnt_hbm, out_hbm,
                 src_v, dst_v, buf0, buf1, acc_sh, cnt_v, sem0, sem1):
    c = lax.axis_index("c")
    s = lax.axis_index("s")

    @pl.when(s < 8)
    def _():
        r0 = s * WRS
        pltpu.sync_copy(y_hbm.at[pl.ds(c * HN + r0, WRS)], acc_sh.at[pl.ds(r0, WRS)])

    pltpu.sync_copy(src_hbm.at[c, s], src_v)
    pltpu.sync_copy(dst_hbm.at[c, s], dst_v)
    pltpu.sync_copy(cnt_hbm.at[c, s], cnt_v)
    plsc.subcore_barrier()
    tc = cnt_v[...][0]

    def start(j, buf, sem):
        pltpu.async_copy(y_hbm.at[src_v.at[j]], buf, sem)

    def finish(j, buf, sem):
        pltpu.make_async_copy(y_hbm.at[src_v.at[j]], buf, sem).wait()
        pltpu.sync_copy(buf, acc_sh.at[dst_v.at[j]], add=True)

    @pl.when(tc > 0)
    def _():
        start(0, buf0, sem0)

    @pl.loop(0, NCH, step=2)
    def _(j):
        @pl.when(j < tc)
        def _():
            @pl.when(j + 1 < tc)
            def _():
                start(j + 1, buf1, sem1)

            finish(j, buf0, sem0)

            @pl.when(j + 2 < tc)
            def _():
                start(j + 2, buf0, sem0)

            @pl.when(j + 1 < tc)
            def _():
                finish(j + 1, buf1, sem1)

    plsc.subcore_barrier()

    @pl.when(s < 8)
    def _():
        r0 = s * WRS
        pltpu.sync_copy(acc_sh.at[pl.ds(r0, WRS)], out_hbm.at[pl.ds(c * HN + r0, WRS)])


def _mm1_body(x_ref, w_ref, o_ref):
    o_ref[0:N] = jnp.dot(x_ref[...], w_ref[...], preferred_element_type=jnp.float32,
                         precision=lax.Precision.HIGHEST)
    o_ref[N:NP] = jnp.zeros((NP - N, D), jnp.float32)


def _scale_body(cnt_ref, xw_ref, dinv_ref, y_ref):
    deg = jnp.sum(cnt_ref[...], axis=1, keepdims=True) + 1.0
    dinv = lax.rsqrt(deg)
    dinv_ref[...] = dinv
    y_ref[...] = xw_ref[...] * dinv


def _mid_body(acc_ref, dinv_ref, b1_ref, w2_ref, y2_ref):
    dinv = dinv_ref[...]
    h = jnp.maximum(acc_ref[...] * dinv + b1_ref[...], 0.0)
    xw2 = jnp.dot(h, w2_ref[...], preferred_element_type=jnp.float32,
                  precision=lax.Precision.HIGHEST)
    y2_ref[...] = xw2 * dinv


def _final_body(acc_ref, dinv_ref, b2_ref, o_ref):
    z = acc_ref[...][:N] * dinv_ref[...][:N] + b2_ref[...]
    z = z - jnp.max(z, axis=1, keepdims=True)
    o_ref[...] = z - jnp.log(jnp.sum(jnp.exp(z), axis=1, keepdims=True))


def kernel(x, edge_index, W1, b1, W2, b2):
    src = edge_index[0].astype(jnp.int32)
    dst = edge_index[1].astype(jnp.int32)
    src_flat = jnp.concatenate([src, jnp.zeros((EP - E,), jnp.int32)])
    dst_flat = jnp.concatenate([dst, jnp.full((EP - E,), NP, jnp.int32)])
    src_flat = src_flat.reshape(NS, ECS)
    dst_flat = dst_flat.reshape(NS, ECS)

    osrc, odst, ocnt, odeg = _prep_kernel(src_flat, dst_flat)
    osrc = osrc.reshape(NC, NS, NCH, CHUNK)
    odst = odst.reshape(NC, NS, NCH, CHUNK)
    counts = jnp.transpose(odeg, (0, 2, 1)).reshape(NP, NS)

    xw1 = pl.pallas_call(
        _mm1_body,
        out_shape=jax.ShapeDtypeStruct((NP, D), jnp.float32),
    )(x, W1)

    dinv, y1 = pl.pallas_call(
        _scale_body,
        out_shape=(jax.ShapeDtypeStruct((NP, 1), jnp.float32),
                   jax.ShapeDtypeStruct((NP, D), jnp.float32)),
    )(counts, xw1)

    acc1 = _prop_kernel(y1, osrc, odst, ocnt)

    y2 = pl.pallas_call(
        _mid_body,
        out_shape=jax.ShapeDtypeStruct((NP, D), jnp.float32),
    )(acc1, dinv, b1.reshape(1, D), W2)

    acc2 = _prop_kernel(y2, osrc, odst, ocnt)

    return pl.pallas_call(
        _final_body,
        out_shape=jax.ShapeDtypeStruct((N, D), jnp.float32),
    )(acc2, dinv, b2.reshape(1, D))


# speedup vs baseline: 32.6544x; 1.0190x over previous
import dataclasses
import functools

import jax
import jax.numpy as jnp
from jax import lax
from jax.experimental import pallas as pl
from jax.experimental.pallas import tpu as pltpu
from jax.experimental.pallas import tpu_sc as plsc

N = 10000
D = 128
E = 320000
NC = 2
NS = 16
NP = 10112
HN = NP // NC
CHUNK = 128
NCH = 158
ECS = NCH * CHUNK
EP = NS * ECS
WRS = 632
DUM = 1024
L = 16

_mesh = plsc.VectorSubcoreMesh(core_axis_name="c", subcore_axis_name="s")

_prep_cp = pltpu.CompilerParams()
if "needs_layout_passes" in pltpu.CompilerParams.__dataclass_fields__:
    _prep_cp = dataclasses.replace(_prep_cp, needs_layout_passes=False)


@functools.partial(
    pl.kernel,
    out_type=(jax.ShapeDtypeStruct((NC, NS, ECS), jnp.int32),
              jax.ShapeDtypeStruct((NC, NS, ECS), jnp.int32),
              jax.ShapeDtypeStruct((NC, NS, L), jnp.int32),
              jax.ShapeDtypeStruct((NC, NS, HN), jnp.float32)),
    mesh=_mesh,
    compiler_params=_prep_cp,
    scratch_types=[
        pltpu.VMEM((ECS,), jnp.int32),
        pltpu.VMEM((ECS,), jnp.int32),
        pltpu.VMEM((ECS + L,), jnp.int32),
        pltpu.VMEM((ECS + L,), jnp.int32),
        pltpu.VMEM((HN,), jnp.float32),
        pltpu.VMEM((L,), jnp.int32),
    ],
)
def _prep_kernel(src_hbm, dst_hbm,
                 osrc_hbm, odst_hbm, ocnt_hbm, odeg_hbm,
                 src_v, dst_v, csrc_v, cdst_v, hist_v, cnt_v):
    c = lax.axis_index("c")
    s = lax.axis_index("s")
    lo = c * HN

    pltpu.sync_copy(src_hbm.at[s], src_v)
    pltpu.sync_copy(dst_hbm.at[s], dst_v)

    base = jax.lax.iota(jnp.int32, L)
    onesv = jnp.ones((L,), jnp.float32)

    @pl.loop(0, ECS + L, step=L)
    def _(i):
        v = base + i
        csrc_v[pl.ds(i, L)] = v & 8191
        cdst_v[pl.ds(i, L)] = HN + (v & (DUM - 1))

    @pl.loop(0, HN, step=L)
    def _(i):
        hist_v[pl.ds(i, L)] = jnp.zeros((L,), jnp.float32)

    def body(i, o):
        d = dst_v[pl.ds(i * L, L)]
        sr = src_v[pl.ds(i * L, L)]
        own = (d >= lo) & (d < lo + HN)
        dl = jnp.where(own, d - lo, 0)
        pref = plsc.cumsum(jnp.where(own, 1, 0))
        pos = jnp.where(own, o + pref - 1, 0)
        plsc.store_scatter(csrc_v, [pos], sr, mask=own)
        plsc.store_scatter(cdst_v, [pos], dl, mask=own)
        plsc.addupdate_scatter(hist_v, [dl], onesv, mask=own)
        return o + jnp.max(pref)

    count = lax.fori_loop(0, ECS // L, body, jnp.int32(0))
    tc = (count + (CHUNK - 1)) >> 7
    cnt_v[...] = jnp.broadcast_to(tc, (L,))

    pltpu.sync_copy(cnt_v, ocnt_hbm.at[c, s])
    pltpu.sync_copy(csrc_v.at[pl.ds(0, ECS)], osrc_hbm.at[c, s])
    pltpu.sync_copy(cdst_v.at[pl.ds(0, ECS)], odst_hbm.at[c, s])
    pltpu.sync_copy(hist_v, odeg_hbm.at[c, s])


@functools.partial(
    pl.kernel,
    out_type=jax.ShapeDtypeStruct((NP, D), jnp.float32),
    mesh=_mesh,
    scratch_types=[
        pltpu.VMEM((NCH, CHUNK), jnp.int32),
        pltpu.VMEM((NCH, CHUNK), jnp.int32),
        pltpu.VMEM((CHUNK, D), jnp.float32),
        pltpu.VMEM((CHUNK, D), jnp.float32),
        pltpu.VMEM_SHARED((HN + DUM, D), jnp.float32),
        pltpu.VMEM((L,), jnp.int32),
        pltpu.SemaphoreType.DMA,
        pltpu.SemaphoreType.DMA,
    ],
)
def _prop_kernel(y_hbm, src_hbm, dst_hbm, cnt_hbm, out_hbm,
                 src_v, dst_v, buf0, buf1, acc_sh, cnt_v, sem0, sem1):
    c = lax.axis_index("c")
    s = lax.axis_index("s")

    @pl.when(s < 8)
    def _():
        r0 = s * WRS
        pltpu.async_copy(y_hbm.at[pl.ds(c * HN + r0, WRS)], acc_sh.at[pl.ds(r0, WRS)], sem0)

    pltpu.async_copy(src_hbm.at[c, s], src_v, sem1)
    pltpu.async_copy(dst_hbm.at[c, s], dst_v, sem1)
    pltpu.async_copy(cnt_hbm.at[c, s], cnt_v, sem1)

    @pl.when(s < 8)
    def _():
        r0 = s * WRS
        pltpu.make_async_copy(y_hbm.at[pl.ds(c * HN + r0, WRS)],
                              acc_sh.at[pl.ds(r0, WRS)], sem0).wait()

    pltpu.make_async_copy(src_hbm.at[c, s], src_v, sem1).wait()
    pltpu.make_async_copy(dst_hbm.at[c, s], dst_v, sem1).wait()
    pltpu.make_async_copy(cnt_hbm.at[c, s], cnt_v, sem1).wait()
    plsc.subcore_barrier()
    tc = cnt_v[...][0]

    def start(j, buf, sem):
        pltpu.async_copy(y_hbm.at[src_v.at[j]], buf, sem)

    def finish(j, buf, sem):
        pltpu.make_async_copy(y_hbm.at[src_v.at[j]], buf, sem).wait()
        pltpu.sync_copy(buf, acc_sh.at[dst_v.at[j]], add=True)

    @pl.when(tc > 0)
    def _():
        start(0, buf0, sem0)

    @pl.loop(0, NCH, step=2)
    def _(j):
        @pl.when(j < tc)
        def _():
            @pl.when(j + 1 < tc)
            def _():
                start(j + 1, buf1, sem1)

            finish(j, buf0, sem0)

            @pl.when(j + 2 < tc)
            def _():
                start(j + 2, buf0, sem0)

            @pl.when(j + 1 < tc)
            def _():
                finish(j + 1, buf1, sem1)

    plsc.subcore_barrier()

    @pl.when(s < 8)
    def _():
        r0 = s * WRS
        pltpu.sync_copy(acc_sh.at[pl.ds(r0, WRS)], out_hbm.at[pl.ds(c * HN + r0, WRS)])


def _mm1_body(x_ref, w_ref, o_ref):
    o_ref[0:N] = jnp.dot(x_ref[...], w_ref[...], preferred_element_type=jnp.float32,
                         precision=lax.Precision.HIGHEST)
    o_ref[N:NP] = jnp.zeros((NP - N, D), jnp.float32)


def _scale_body(cnt_ref, xw_ref, dinv_ref, y_ref):
    deg = jnp.sum(cnt_ref[...], axis=1, keepdims=True) + 1.0
    dinv = lax.rsqrt(deg)
    dinv_ref[...] = dinv
    y_ref[...] = xw_ref[...] * dinv


def _mid_body(acc_ref, dinv_ref, b1_ref, w2_ref, y2_ref):
    dinv = dinv_ref[...]
    h = jnp.maximum(acc_ref[...] * dinv + b1_ref[...], 0.0)
    xw2 = jnp.dot(h, w2_ref[...], preferred_element_type=jnp.float32,
                  precision=lax.Precision.HIGHEST)
    y2_ref[...] = xw2 * dinv


def _final_body(acc_ref, dinv_ref, b2_ref, o_ref):
    z = acc_ref[...][:N] * dinv_ref[...][:N] + b2_ref[...]
    z = z - jnp.max(z, axis=1, keepdims=True)
    o_ref[...] = z - jnp.log(jnp.sum(jnp.exp(z), axis=1, keepdims=True))


def kernel(x, edge_index, W1, b1, W2, b2):
    src = edge_index[0].astype(jnp.int32)
    dst = edge_index[1].astype(jnp.int32)
    src_flat = jnp.concatenate([src, jnp.zeros((EP - E,), jnp.int32)])
    dst_flat = jnp.concatenate([dst, jnp.full((EP - E,), NP, jnp.int32)])
    src_flat = src_flat.reshape(NS, ECS)
    dst_flat = dst_flat.reshape(NS, ECS)

    osrc, odst, ocnt, odeg = _prep_kernel(src_flat, dst_flat)
    osrc = osrc.reshape(NC, NS, NCH, CHUNK)
    odst = odst.reshape(NC, NS, NCH, CHUNK)
    counts = jnp.transpose(odeg, (0, 2, 1)).reshape(NP, NS)

    xw1 = pl.pallas_call(
        _mm1_body,
        out_shape=jax.ShapeDtypeStruct((NP, D), jnp.float32),
    )(x, W1)

    dinv, y1 = pl.pallas_call(
        _scale_body,
        out_shape=(jax.ShapeDtypeStruct((NP, 1), jnp.float32),
                   jax.ShapeDtypeStruct((NP, D), jnp.float32)),
    )(counts, xw1)

    acc1 = _prop_kernel(y1, osrc, odst, ocnt)

    y2 = pl.pallas_call(
        _mid_body,
        out_shape=jax.ShapeDtypeStruct((NP, D), jnp.float32),
    )(acc1, dinv, b1.reshape(1, D), W2)

    acc2 = _prop_kernel(y2, osrc, odst, ocnt)

    return pl.pallas_call(
        _final_body,
        out_shape=jax.ShapeDtypeStruct((N, D), jnp.float32),
    )(acc2, dinv, b2.reshape(1, D))
